# baseline (speedup 1.0000x reference)
import jax
import jax.numpy as jnp
from jax import lax
from jax.experimental import pallas as pl
from jax.experimental.pallas import tpu as pltpu
from jax.experimental.pallas import tpu_sc as plsc

_B = 16384
_DIM = 32
_NC, _NS, _L = 2, 16, 16
_NW = _NC * _NS
_RPW = _B // _NW
_CHUNK = 128
_NCHUNK = _RPW // _CHUNK


def _tec_body(h_ids, r_typ, t_ids, ent_emb, rel_emb, out,
              idx_h, idx_r, idx_t, h_rows, r_rows, t_rows, tbuf, out_v, sem):
    wid = lax.axis_index("s") * _NC + lax.axis_index("c")
    base = wid * _RPW

    pltpu.sync_copy(h_ids.at[pl.ds(base, _RPW)], idx_h)
    pltpu.sync_copy(r_typ.at[pl.ds(base, _RPW)], idx_r)
    pltpu.sync_copy(t_ids.at[pl.ds(base, _RPW)], idx_t)

    copies = []
    for c in range(_NCHUNK):
        sl = pl.ds(c * _CHUNK, _CHUNK)
        copies.append(pltpu.async_copy(ent_emb.at[idx_h.at[sl]], h_rows.at[sl], sem))
        copies.append(pltpu.async_copy(rel_emb.at[idx_r.at[sl]], r_rows.at[sl], sem))
        copies.append(pltpu.async_copy(ent_emb.at[idx_t.at[sl]], t_rows.at[sl], sem))
    for cp in copies:
        cp.wait()

    lane = lax.iota(jnp.int32, _L)
    scat_base = lane * _RPW

    def row_body(i, carry):
        h0 = h_rows[i, pl.ds(0, _L)]
        h1 = h_rows[i, pl.ds(_L, _L)]
        r0 = r_rows[i, pl.ds(0, _L)]
        r1 = r_rows[i, pl.ds(_L, _L)]
        t0 = t_rows[i, pl.ds(0, _L)]
        t1 = t_rows[i, pl.ds(_L, _L)]
        d0 = (h0 + r0) - t0
        d1 = (h1 + r1) - t1
        s = d0 * d0 + d1 * d1
        plsc.store_scatter(tbuf, [scat_base + i], s)
        return carry

    lax.fori_loop(0, _RPW, row_body, 0)

    def out_body(cblk, carry):
        off = cblk * _L
        acc = tbuf[pl.ds(off, _L)]
        for l in range(1, _L):
            acc = acc + tbuf[pl.ds(l * _RPW + off, _L)]
        x = jnp.maximum(acc, jnp.float32(1e-30))
        bits = plsc.bitcast(x, jnp.int32)
        bits = jnp.int32(0x5F3759DF) - lax.shift_right_arithmetic(bits, 1)
        y = plsc.bitcast(bits, jnp.float32)
        for _ in range(3):
            y = y * (jnp.float32(1.5) - jnp.float32(0.5) * x * y * y)
        out_v[pl.ds(off, _L)] = acc * y
        return carry

    lax.fori_loop(0, _RPW // _L, out_body, 0)

    pltpu.sync_copy(out_v, out.at[pl.ds(base, _RPW)])


def kernel(h_ids, r_typ, t_ids, ent_emb, rel_emb):
    mesh = plsc.VectorSubcoreMesh(core_axis_name="c", subcore_axis_name="s",
                                  num_cores=_NC, num_subcores=_NS)
    f = pl.kernel(
        _tec_body,
        out_type=jax.ShapeDtypeStruct((_B,), jnp.float32),
        mesh=mesh,
        compiler_params=pltpu.CompilerParams(needs_layout_passes=False,
                                             use_tc_tiling_on_sc=False),
        scratch_types=[
            pltpu.VMEM((_RPW,), jnp.int32),
            pltpu.VMEM((_RPW,), jnp.int32),
            pltpu.VMEM((_RPW,), jnp.int32),
            pltpu.VMEM((_RPW, _DIM), jnp.float32),
            pltpu.VMEM((_RPW, _DIM), jnp.float32),
            pltpu.VMEM((_RPW, _DIM), jnp.float32),
            pltpu.VMEM((_L * _RPW,), jnp.float32),
            pltpu.VMEM((_RPW,), jnp.float32),
            pltpu.SemaphoreType.DMA,
        ],
    )
    return f(h_ids.astype(jnp.int32), r_typ.astype(jnp.int32),
             t_ids.astype(jnp.int32), ent_emb, rel_emb)
